# baseline (speedup 1.0000x reference)
import functools

import jax
import jax.numpy as jnp
from jax import lax
from jax.experimental import pallas as pl
from jax.experimental.pallas import tpu as pltpu
from jax.experimental.pallas import tpu_sc as plsc

SCALE_ = 8.0
_BAND = 128
_D = 64


def _make_emb(n_bands: int, seq: int):
  info = plsc.get_sparse_core_info()
  nc, ns, nl = info.num_cores, info.num_subcores, info.num_lanes
  nw = nc * ns
  bands_w = n_bands // nw
  tjn = n_bands // seq
  assert bands_w % 2 == 0 and nl == 16 and _D == 64

  mesh = plsc.VectorSubcoreMesh(core_axis_name="c", subcore_axis_name="s")

  @functools.partial(
      pl.kernel,
      mesh=mesh,
      compiler_params=pltpu.CompilerParams(use_tc_tiling_on_sc=False,
                                           needs_layout_passes=False,
                                           disable_bounds_checks=True),
      out_type=jax.ShapeDtypeStruct((seq, 8, tjn, 8, _BAND), jnp.float32),
      scratch_types=[
          pltpu.VMEM((bands_w, _BAND), jnp.int32),
          pltpu.VMEM((_BAND, _D), jnp.float32),
          pltpu.VMEM((_BAND, _D), jnp.float32),
          pltpu.VMEM((1, _BAND, _D), jnp.float32),
          pltpu.VMEM((1, _BAND, _D), jnp.float32),
          pltpu.VMEM((8, 1, 8, _BAND), jnp.float32),
          pltpu.SemaphoreType.DMA,
          pltpu.SemaphoreType.DMA,
          pltpu.SemaphoreType.DMA,
          pltpu.SemaphoreType.DMA,
      ],
  )
  def emb(idx_hbm, table_hbm, out_hbm, idx_v, g0, g1, t0, t1, dummy_v,
          sem_g0, sem_g1, sem_w0, sem_w1):
    wid = lax.axis_index("s") * nc + lax.axis_index("c")
    band0 = wid * bands_w
    bufs = ((g0, t0, sem_g0, sem_w0), (g1, t1, sem_g1, sem_w1))

    pltpu.sync_copy(idx_hbm.at[pl.ds(band0, bands_w)], idx_v)

    def fire(q, g_v, sem):
      pltpu.async_copy(table_hbm.at[idx_v.at[q]], g_v, sem)

    def drain_g(g_v, sem):
      pltpu.make_async_copy(table_hbm.at[pl.ds(0, _BAND)], g_v, sem).wait()

    def drain_w(sem):
      pltpu.make_async_copy(
          out_hbm.at[0, :, pl.ds(0, 1)], dummy_v, sem).wait()

    def scale_copy(g_v, t_v):
      @plsc.parallel_loop(0, _BAND)
      def rstep(r):
        for g in range(_D // nl):
          sl = pl.ds(g * nl, nl)
          t_v[0, r, sl] = g_v[r, sl] * SCALE_

    fire(0, g0, sem_g0)
    fire(1, g1, sem_g1)

    def chunk_pair(k, carry):
      for b, (g_v, t_v, sem_g, sem_w) in enumerate(bufs):
        q = 2 * k + b
        gb = band0 + q
        s = gb // tjn
        tj = gb % tjn
        drain_g(g_v, sem_g)

        @pl.when(k > 0)
        def _t_free():
          drain_w(sem_w)

        scale_copy(g_v, t_v)

        @pl.when(k < bands_w // 2 - 1)
        def _prefetch():
          fire(q + 2, g_v, sem_g)

        for d in range(_D):
          pltpu.async_copy(
              t_v.at[:, :, d], out_hbm.at[s, d // 8, tj, pl.ds(d % 8, 1)],
              sem_w)
      return carry

    lax.fori_loop(0, bands_w // 2, chunk_pair, 0)
    drain_w(sem_w0)
    drain_w(sem_w1)

  return emb


def kernel(x, table):
  b, s = x.shape
  vocab, d = table.shape
  n_bands = (b * s) // _BAND
  tjn = b // _BAND
  idx2d = x.T.reshape(n_bands, _BAND)
  emb = _make_emb(n_bands, s)
  out5 = emb(idx2d, table)
  return out5.transpose(2, 4, 0, 1, 3).reshape(b, s, d)


# speedup vs baseline: 92.4476x; 92.4476x over previous
import functools

import jax
import jax.numpy as jnp
from jax import lax
from jax.experimental import pallas as pl
from jax.experimental.pallas import tpu as pltpu
from jax.experimental.pallas import tpu_sc as plsc

SCALE_ = 8.0

_IDX_MINOR = 128
_GATHERS_PER_CHUNK = 5
_CHUNK = _IDX_MINOR * _GATHERS_PER_CHUNK


def _make_emb(n_idx_rows: int, vocab: int, d: int):
  info = plsc.get_sparse_core_info()
  nc, ns, nl = info.num_cores, info.num_subcores, info.num_lanes
  nw = nc * ns
  total = n_idx_rows * _IDX_MINOR
  per_w = total // nw
  idx_rows_w = per_w // _IDX_MINOR
  n_chunks = per_w // _CHUNK
  assert per_w % _CHUNK == 0 and n_chunks % 2 == 0 and d % nl == 0

  mesh = plsc.VectorSubcoreMesh(core_axis_name="c", subcore_axis_name="s")

  @functools.partial(
      pl.kernel,
      mesh=mesh,
      compiler_params=pltpu.CompilerParams(use_tc_tiling_on_sc=False),
      out_type=jax.ShapeDtypeStruct((total, d), jnp.float32),
      scratch_types=[
          pltpu.VMEM((idx_rows_w, _IDX_MINOR), jnp.int32),
          pltpu.VMEM((_CHUNK, d), jnp.float32),
          pltpu.VMEM((_CHUNK, d), jnp.float32),
          pltpu.SemaphoreType.DMA,
          pltpu.SemaphoreType.DMA,
          pltpu.SemaphoreType.DMA,
          pltpu.SemaphoreType.DMA,
      ],
  )
  def emb(idx_hbm, table_hbm, out_hbm, idx_v, rows0, rows1,
          sem_g0, sem_g1, sem_w0, sem_w1):
    wid = lax.axis_index("s") * nc + lax.axis_index("c")
    out_row0 = wid * per_w
    bufs = ((rows0, sem_g0, sem_w0), (rows1, sem_g1, sem_w1))

    pltpu.sync_copy(idx_hbm.at[pl.ds(wid * idx_rows_w, idx_rows_w)], idx_v)

    def fire(c, rows, sem):
      for j in range(_GATHERS_PER_CHUNK):
        pltpu.async_copy(
            table_hbm.at[idx_v.at[c * _GATHERS_PER_CHUNK + j]],
            rows.at[pl.ds(j * _IDX_MINOR, _IDX_MINOR)],
            sem,
        )

    def drain(rows, sem):
      pltpu.make_async_copy(out_hbm.at[pl.ds(0, _CHUNK)], rows, sem).wait()

    fire(0, rows0, sem_g0)
    fire(1, rows1, sem_g1)

    def chunk_pair(k, carry):
      for b, (rows, sem_g, sem_w) in enumerate(bufs):
        c = 2 * k + b
        drain(rows, sem_g)

        def scale_row(r, carry2):
          for j in range(d // nl):
            s = pl.ds(j * nl, nl)
            rows[r, s] = rows[r, s] * SCALE_
          return carry2

        lax.fori_loop(0, _CHUNK, scale_row, 0)
        pltpu.async_copy(rows, out_hbm.at[pl.ds(out_row0 + c * _CHUNK, _CHUNK)],
                         sem_w)

      @pl.when(k < n_chunks // 2 - 1)
      def _prefetch():
        for b, (rows, sem_g, sem_w) in enumerate(bufs):
          drain(rows, sem_w)
          fire(2 * k + b + 2, rows, sem_g)

      return carry

    lax.fori_loop(0, n_chunks // 2, chunk_pair, 0)
    drain(rows0, sem_w0)
    drain(rows1, sem_w1)

  return emb


def kernel(x, table):
  b, s = x.shape
  vocab, d = table.shape
  total = b * s
  idx2d = x.reshape(total // _IDX_MINOR, _IDX_MINOR)
  emb = _make_emb(total // _IDX_MINOR, vocab, d)
  out = emb(idx2d, table)
  return out.reshape(b, s, d)
